# baseline (speedup 1.0000x reference)
import jax
import jax.numpy as jnp
from jax.experimental import pallas as pl
from jax.experimental.pallas import tpu as pltpu

N = 640000
D = 16
HIST = 100
ALPHA = 0.4
BETA = 0.4
GAMMA = 0.2
DECAY = 0.99

B = 5120
NB = N // B


def _pass1(g_ref, s_ref, t_ref, ge_ref, fe_ref, hist_ref, w_ref,
           nge_ref, nfe_ref, pmax_ref, nhist_ref):
    nge = DECAY * ge_ref[...] + (1.0 - DECAY) * jnp.abs(g_ref[...])
    nge_ref[...] = nge
    pmax_ref[...] = jnp.full((128,), jnp.max(nge), dtype=jnp.float32)

    s = s_ref[...].T
    t = t_ref[...].T
    sum_s = jnp.sum(s, axis=0)
    sum_t = jnp.sum(t, axis=0)
    sum_st = jnp.sum(s * t, axis=0)
    sum_ss = jnp.sum(s * s, axis=0)
    sum_tt = jnp.sum(t * t, axis=0)
    cov = sum_st - sum_s * sum_t * (1.0 / D)
    var_s = sum_ss - sum_s * sum_s * (1.0 / D)
    var_t = sum_tt - sum_t * sum_t * (1.0 / D)
    corr = cov / ((jnp.sqrt(var_s) + 1e-6) * (jnp.sqrt(var_t) + 1e-6))
    nfe_ref[...] = DECAY * fe_ref[...] + (1.0 - DECAY) * jnp.abs(corr)

    nhist_ref[...] = hist_ref[...]
    nhist_ref[0:1, :] = jnp.reshape(w_ref[...], (1, B))


def _pass2(pmax_ref, nge_ref, nfe_ref, u_ref):
    m = jnp.max(pmax_ref[...])
    u_ref[...] = (ALPHA / (m + 1e-6)) * nge_ref[...] \
        + (BETA * nfe_ref[...] + GAMMA)


def kernel(gradients, source_activations, target_activations, weights,
           gradient_ema, flow_ema, weight_history):
    row = lambda i: (i,)
    nge, nfe, pmax, nhist = pl.pallas_call(
        _pass1,
        grid=(NB,),
        in_specs=[
            pl.BlockSpec((B,), row),
            pl.BlockSpec((B, D), lambda i: (i, 0)),
            pl.BlockSpec((B, D), lambda i: (i, 0)),
            pl.BlockSpec((B,), row),
            pl.BlockSpec((B,), row),
            pl.BlockSpec((HIST, B), lambda i: (0, i)),
            pl.BlockSpec((B,), row),
        ],
        out_specs=[
            pl.BlockSpec((B,), row),
            pl.BlockSpec((B,), row),
            pl.BlockSpec((128,), row),
            pl.BlockSpec((HIST, B), lambda i: (0, i)),
        ],
        out_shape=[
            jax.ShapeDtypeStruct((N,), jnp.float32),
            jax.ShapeDtypeStruct((N,), jnp.float32),
            jax.ShapeDtypeStruct((NB * 128,), jnp.float32),
            jax.ShapeDtypeStruct((HIST, N), jnp.float32),
        ],
    )(gradients, source_activations, target_activations,
      gradient_ema, flow_ema, weight_history, weights)

    utility = pl.pallas_call(
        _pass2,
        grid=(NB,),
        in_specs=[
            pl.BlockSpec((NB * 128,), lambda i: (0,)),
            pl.BlockSpec((B,), row),
            pl.BlockSpec((B,), row),
        ],
        out_specs=pl.BlockSpec((B,), row),
        out_shape=jax.ShapeDtypeStruct((N,), jnp.float32),
    )(pmax, nge, nfe)

    return (utility, nge, nfe, nhist)


# speedup vs baseline: 3.7660x; 3.7660x over previous
import jax
import jax.numpy as jnp
from jax.experimental import pallas as pl
from jax.experimental.pallas import tpu as pltpu

N = 640000
D = 16
HIST = 100
ALPHA = 0.4
BETA = 0.4
GAMMA = 0.2
DECAY = 0.99

B = 25600
NB = N // B


def _body(g_ref, s_ref, t_ref, ge_ref, fe_ref, hist_ref, w_ref,
          nge_ref, nfe_ref, u_ref, nhist_ref, nge_s, nfe_s, smax):
    i = pl.program_id(0)

    @pl.when(i < NB)
    def _phase1():
        nge = DECAY * ge_ref[...] + (1.0 - DECAY) * jnp.abs(g_ref[...])
        nge_ref[...] = nge
        nge_s[pl.ds(i * B, B)] = nge
        bmax = jnp.max(nge)
        prev = jnp.where(i == 0, 0.0, smax[0])
        smax[0] = jnp.maximum(prev, bmax)

        s = s_ref[...]
        t = t_ref[...]
        sum_s = jnp.sum(s, axis=0)
        sum_t = jnp.sum(t, axis=0)
        sum_st = jnp.sum(s * t, axis=0)
        sum_ss = jnp.sum(s * s, axis=0)
        sum_tt = jnp.sum(t * t, axis=0)
        cov = sum_st - sum_s * sum_t * (1.0 / D)
        var_s = sum_ss - sum_s * sum_s * (1.0 / D)
        var_t = sum_tt - sum_t * sum_t * (1.0 / D)
        corr = cov / ((jnp.sqrt(var_s) + 1e-6) * (jnp.sqrt(var_t) + 1e-6))
        nfe = DECAY * fe_ref[...] + (1.0 - DECAY) * jnp.abs(corr)
        nfe_ref[...] = nfe
        nfe_s[pl.ds(i * B, B)] = nfe

        nhist_ref[...] = hist_ref[...]
        nhist_ref[0:1, :] = jnp.reshape(w_ref[...], (1, B))

    @pl.when(i >= NB)
    def _phase2():
        j = i - NB
        m = smax[0]
        u_ref[...] = (ALPHA / (m + 1e-6)) * nge_s[pl.ds(j * B, B)] \
            + (BETA * nfe_s[pl.ds(j * B, B)] + GAMMA)


def kernel(gradients, source_activations, target_activations, weights,
           gradient_ema, flow_ema, weight_history):
    sT = source_activations.T
    tT = target_activations.T

    clamp = lambda i: (jnp.minimum(i, NB - 1),)
    clamp2 = lambda i: (0, jnp.minimum(i, NB - 1))
    nge, nfe, utility, nhist = pl.pallas_call(
        _body,
        grid=(2 * NB,),
        in_specs=[
            pl.BlockSpec((B,), clamp),
            pl.BlockSpec((D, B), clamp2),
            pl.BlockSpec((D, B), clamp2),
            pl.BlockSpec((B,), clamp),
            pl.BlockSpec((B,), clamp),
            pl.BlockSpec((HIST, B), clamp2),
            pl.BlockSpec((B,), clamp),
        ],
        out_specs=[
            pl.BlockSpec((B,), clamp),
            pl.BlockSpec((B,), clamp),
            pl.BlockSpec((B,), lambda i: (jnp.maximum(i - NB, 0),)),
            pl.BlockSpec((HIST, B), clamp2),
        ],
        out_shape=[
            jax.ShapeDtypeStruct((N,), jnp.float32),
            jax.ShapeDtypeStruct((N,), jnp.float32),
            jax.ShapeDtypeStruct((N,), jnp.float32),
            jax.ShapeDtypeStruct((HIST, N), jnp.float32),
        ],
        scratch_shapes=[
            pltpu.VMEM((N,), jnp.float32),
            pltpu.VMEM((N,), jnp.float32),
            pltpu.SMEM((1,), jnp.float32),
        ],
    )(gradients, sT, tT, gradient_ema, flow_ema, weight_history, weights)

    return (utility, nge, nfe, nhist)
